# Initial kernel scaffold; baseline (speedup 1.0000x reference)
import jax
import jax.numpy as jnp
from jax.experimental import pallas as pl


def kernel(x, edge_index, batch, W1, b1, g1, be1, W2, b2, g2, be2, Wl1, bl1, g3, be3, Wl2, bl2):
    raise NotImplementedError("write your pallas kernel here")


# speedup vs baseline: 7.5820x; 7.5820x over previous
import jax
import jax.numpy as jnp
from jax import lax
from jax.experimental import pallas as pl
from jax.experimental.pallas import tpu as pltpu
from jax.experimental.pallas import tpu_sc as plsc

N = 10000
E = 320000
G = 128
D = 128
C = 16
EPS = 1e-5

NW = 32
N_PAD = 10240
CHUNK = 128
E_PAD = 327680
CPW = E_PAD // (NW * CHUNK)
EPW = E_PAD // NW
RPT = N_PAD // 16
SINK = N
RPW = N_PAD // NW
GP = G + 1

BLK = 400
NB = N // BLK

_mesh = plsc.VectorSubcoreMesh(core_axis_name="c", subcore_axis_name="s")
_sc_params = pltpu.CompilerParams(needs_layout_passes=False)


def _sc_degree_body(dst_hbm, out_hbm, dst_v, acc_v):
    c = lax.axis_index("c")
    s = lax.axis_index("s")
    wid = s * 2 + c
    pltpu.sync_copy(dst_hbm.at[wid], dst_v)
    zero16 = jnp.zeros((16,), jnp.float32)

    def zbody(i, carry):
        acc_v[pl.ds(i * 16, 16)] = zero16
        return carry

    lax.fori_loop(0, N_PAD // 16, zbody, 0)
    one16 = jnp.ones((16,), jnp.float32)

    def body(i, carry):
        idx = dst_v[pl.ds(i * 16, 16)]
        plsc.addupdate_scatter(acc_v, [idx], one16)
        return carry

    lax.fori_loop(0, EPW // 16, body, 0)
    pltpu.sync_copy(acc_v, out_hbm.at[wid])


_sc_degree = pl.kernel(
    _sc_degree_body,
    out_type=jax.ShapeDtypeStruct((NW, N_PAD), jnp.float32),
    mesh=_mesh,
    compiler_params=_sc_params,
    scratch_types=[
        pltpu.VMEM((EPW,), jnp.int32),
        pltpu.VMEM((N_PAD,), jnp.float32),
    ],
)


def _sc_propagate_body(hs_hbm, src_hbm, dst_hbm, zero_hbm, out_hbm,
                       src_v, dst_v, rows_v, zbuf_v, acc_sh, sem):
    c = lax.axis_index("c")
    s = lax.axis_index("s")
    wid = s * 2 + c
    pltpu.sync_copy(zero_hbm, zbuf_v)

    def zbody(k, carry):
        pltpu.sync_copy(zbuf_v, acc_sh.at[pl.ds(s * RPT + k * CHUNK, CHUNK)])
        return carry

    lax.fori_loop(0, RPT // CHUNK, zbody, 0)
    plsc.subcore_barrier()

    def body(j, carry):
        pltpu.sync_copy(src_hbm.at[wid, j], src_v)
        pltpu.sync_copy(dst_hbm.at[wid, j], dst_v)
        pltpu.async_copy(hs_hbm.at[src_v], rows_v, sem).wait()
        pltpu.sync_copy(rows_v, acc_sh.at[dst_v], add=True)
        return carry

    lax.fori_loop(0, CPW, body, 0)
    plsc.subcore_barrier()

    def obody(k, carry):
        pltpu.sync_copy(acc_sh.at[pl.ds(s * RPT + k * CHUNK, CHUNK)], zbuf_v)
        pltpu.sync_copy(zbuf_v, out_hbm.at[c, s, pl.ds(k * CHUNK, CHUNK)])
        return carry

    lax.fori_loop(0, RPT // CHUNK, obody, 0)


_sc_propagate = pl.kernel(
    _sc_propagate_body,
    out_type=jax.ShapeDtypeStruct((2, 16, RPT, D), jnp.float32),
    mesh=_mesh,
    compiler_params=_sc_params,
    scratch_types=[
        pltpu.VMEM((CHUNK,), jnp.int32),
        pltpu.VMEM((CHUNK,), jnp.int32),
        pltpu.VMEM((CHUNK, D), jnp.float32),
        pltpu.VMEM((CHUNK, D), jnp.float32),
        pltpu.VMEM_SHARED((N_PAD, D), jnp.float32),
        pltpu.SemaphoreType.DMA,
    ],
)


def _sc_segmax_body(t_hbm, b_hbm, sc_hbm, sh_hbm, out_hbm,
                    t_v, b_v, sc_v, sh_v, acc_v):
    c = lax.axis_index("c")
    s = lax.axis_index("s")
    wid = s * 2 + c
    pltpu.sync_copy(t_hbm.at[wid], t_v)
    pltpu.sync_copy(b_hbm.at[wid], b_v)
    pltpu.sync_copy(sc_hbm, sc_v)
    pltpu.sync_copy(sh_hbm, sh_v)
    ninf = jnp.full((16,), -jnp.inf, jnp.float32)

    def zbody(i, carry):
        acc_v[pl.ds(i * 16, 16)] = ninf
        return carry

    lax.fori_loop(0, GP * D // 16, zbody, 0)
    lanes = lax.iota(jnp.int32, 16)

    def group(i, carry):
        gv = b_v[pl.ds(i * 16, 16)]
        for k in range(16):
            gi = jnp.max(jnp.where(lanes == k, gv, 0))
            for j in range(8):
                off = gi * D + j * 16
                tv = t_v[pl.ds(i * (16 * D) + k * D + j * 16, 16)]
                u = jnp.maximum(
                    tv * sc_v[pl.ds(j * 16, 16)] + sh_v[pl.ds(j * 16, 16)],
                    0.0)
                acc_v[pl.ds(off, 16)] = jnp.maximum(acc_v[pl.ds(off, 16)], u)
        return carry

    lax.fori_loop(0, RPW // 16, group, 0)
    pltpu.sync_copy(acc_v, out_hbm.at[wid])


_sc_segmax = pl.kernel(
    _sc_segmax_body,
    out_type=jax.ShapeDtypeStruct((NW, GP * D), jnp.float32),
    mesh=_mesh,
    compiler_params=_sc_params,
    scratch_types=[
        pltpu.VMEM((RPW * D,), jnp.float32),
        pltpu.VMEM((RPW,), jnp.int32),
        pltpu.VMEM((D,), jnp.float32),
        pltpu.VMEM((D,), jnp.float32),
        pltpu.VMEM((GP * D,), jnp.float32),
    ],
)


def _tc_scale_mm_body(x_ref, w_ref, deg_ref, hs_ref, dinv_ref):
    h = jnp.dot(x_ref[...], w_ref[...], preferred_element_type=jnp.float32)
    deg = jnp.sum(deg_ref[...], axis=1) + 1.0
    dinv = lax.rsqrt(deg)[:, None]
    hs_ref[...] = h * dinv
    dinv_ref[...] = jnp.broadcast_to(dinv, (BLK, D))


def _tc_scale_mm(x, w, deg_parts_t):
    return pl.pallas_call(
        _tc_scale_mm_body,
        grid=(NB,),
        in_specs=[
            pl.BlockSpec((BLK, D), lambda i: (i, 0)),
            pl.BlockSpec((D, D), lambda i: (0, 0)),
            pl.BlockSpec((BLK, NW), lambda i: (i, 0)),
        ],
        out_specs=[
            pl.BlockSpec((BLK, D), lambda i: (i, 0)),
            pl.BlockSpec((BLK, D), lambda i: (i, 0)),
        ],
        out_shape=[
            jax.ShapeDtypeStruct((N, D), jnp.float32),
            jax.ShapeDtypeStruct((N, D), jnp.float32),
        ],
    )(x, w, deg_parts_t)


def _tc_combine_body(acc_ref, hs_ref, dinv_ref, b_ref, g_ref, be_ref,
                     t_ref, sum_ref, sq_ref, scale_ref, shift_ref):
    t = dinv_ref[...] * (acc_ref[0] + acc_ref[1] + hs_ref[...]) + b_ref[...]
    t_ref[...] = t

    @pl.when(pl.program_id(0) == 0)
    def _():
        sum_ref[...] = jnp.zeros_like(sum_ref)
        sq_ref[...] = jnp.zeros_like(sq_ref)

    sum_ref[...] += jnp.sum(t, axis=0, keepdims=True)
    sq_ref[...] += jnp.sum(t * t, axis=0, keepdims=True)
    mean = sum_ref[...] / N
    var = sq_ref[...] / N - mean * mean
    scale = g_ref[...] * lax.rsqrt(var + EPS)
    scale_ref[...] = scale
    shift_ref[...] = be_ref[...] - mean * scale


def _tc_combine(acc, hs, dinv, b, g, be):
    return pl.pallas_call(
        _tc_combine_body,
        grid=(NB,),
        in_specs=[
            pl.BlockSpec((2, BLK, D), lambda i: (0, i, 0)),
            pl.BlockSpec((BLK, D), lambda i: (i, 0)),
            pl.BlockSpec((BLK, D), lambda i: (i, 0)),
            pl.BlockSpec((1, D), lambda i: (0, 0)),
            pl.BlockSpec((1, D), lambda i: (0, 0)),
            pl.BlockSpec((1, D), lambda i: (0, 0)),
        ],
        out_specs=[
            pl.BlockSpec((BLK, D), lambda i: (i, 0)),
            pl.BlockSpec((1, D), lambda i: (0, 0)),
            pl.BlockSpec((1, D), lambda i: (0, 0)),
            pl.BlockSpec((1, D), lambda i: (0, 0)),
            pl.BlockSpec((1, D), lambda i: (0, 0)),
        ],
        out_shape=[
            jax.ShapeDtypeStruct((N_PAD, D), jnp.float32),
            jax.ShapeDtypeStruct((1, D), jnp.float32),
            jax.ShapeDtypeStruct((1, D), jnp.float32),
            jax.ShapeDtypeStruct((1, D), jnp.float32),
            jax.ShapeDtypeStruct((1, D), jnp.float32),
        ],
    )(acc, hs, dinv, b.reshape(1, D), g.reshape(1, D), be.reshape(1, D))


def _tc_bn_mm_body(t_ref, scale_ref, shift_ref, w_ref, dinv_ref, out_ref):
    u = jnp.maximum(t_ref[...] * scale_ref[...] + shift_ref[...], 0.0)
    out_ref[...] = jnp.dot(u, w_ref[...],
                           preferred_element_type=jnp.float32) * dinv_ref[...]


def _tc_bn_mm(t, scale, shift, w, dinv):
    return pl.pallas_call(
        _tc_bn_mm_body,
        grid=(NB,),
        in_specs=[
            pl.BlockSpec((BLK, D), lambda i: (i, 0)),
            pl.BlockSpec((1, D), lambda i: (0, 0)),
            pl.BlockSpec((1, D), lambda i: (0, 0)),
            pl.BlockSpec((D, D), lambda i: (0, 0)),
            pl.BlockSpec((BLK, D), lambda i: (i, 0)),
        ],
        out_specs=pl.BlockSpec((BLK, D), lambda i: (i, 0)),
        out_shape=jax.ShapeDtypeStruct((N, D), jnp.float32),
    )(t, scale, shift, w, dinv)


def _tc_head_body(p_ref, wl1_ref, bl1_ref, g_ref, be_ref, wl2_ref, bl2_ref,
                  out_ref):
    p = p_ref[...]
    pm = p[0]
    for w in range(1, NW):
        pm = jnp.maximum(pm, p[w])
    pooled = pm[:G, :]
    z = jnp.dot(pooled, wl1_ref[...],
                preferred_element_type=jnp.float32) + bl1_ref[...]
    mean = jnp.mean(z, axis=0, keepdims=True)
    var = jnp.mean((z - mean) * (z - mean), axis=0, keepdims=True)
    zb = jnp.maximum((z - mean) * lax.rsqrt(var + EPS) * g_ref[...]
                     + be_ref[...], 0.0)
    z2 = jnp.dot(zb, wl2_ref[...],
                 preferred_element_type=jnp.float32) + bl2_ref[...]
    m = jnp.max(z2, axis=1, keepdims=True)
    lse = jnp.log(jnp.sum(jnp.exp(z2 - m), axis=1, keepdims=True)) + m
    out_ref[...] = z2 - lse


def _tc_head(parts, wl1, bl1, g3, be3, wl2, bl2):
    return pl.pallas_call(
        _tc_head_body,
        out_shape=jax.ShapeDtypeStruct((G, C), jnp.float32),
    )(parts, wl1, bl1.reshape(1, D), g3.reshape(1, D), be3.reshape(1, D),
      wl2, bl2.reshape(1, C))


def kernel(x, edge_index, batch, W1, b1, g1, be1, W2, b2, g2, be2,
           Wl1, bl1, g3, be3, Wl2, bl2):
    src = edge_index[0].astype(jnp.int32)
    dst = edge_index[1].astype(jnp.int32)
    pad = E_PAD - E
    src_p = jnp.concatenate([src, jnp.zeros((pad,), jnp.int32)])
    dst_p = jnp.concatenate([dst, jnp.full((pad,), SINK, jnp.int32)])
    src3 = src_p.reshape(NW, CPW, CHUNK)
    dst3 = dst_p.reshape(NW, CPW, CHUNK)
    dst2 = dst_p.reshape(NW, EPW)
    zeros = jnp.zeros((CHUNK, D), jnp.float32)
    batch2 = jnp.concatenate([batch.astype(jnp.int32),
                              jnp.full((N_PAD - N,), G, jnp.int32)])
    batch2 = batch2.reshape(NW, RPW)

    deg_parts = _sc_degree(dst2)
    hs, dinv = _tc_scale_mm(x, W1, deg_parts.T)

    acc1 = _sc_propagate(hs, src3, dst3, zeros).reshape(2, N_PAD, D)
    t1, _, _, scale1, shift1 = _tc_combine(acc1, hs, dinv, b1, g1, be1)
    h2s = _tc_bn_mm(t1, scale1, shift1, W2, dinv)

    acc2 = _sc_propagate(h2s, src3, dst3, zeros).reshape(2, N_PAD, D)
    t2, _, _, scale2, shift2 = _tc_combine(acc2, h2s, dinv, b2, g2, be2)
    parts = _sc_segmax(t2.reshape(NW, RPW * D), batch2,
                       scale2.reshape(D), shift2.reshape(D))

    return _tc_head(parts.reshape(NW, GP, D), Wl1, bl1, g3, be3, Wl2, bl2)


# speedup vs baseline: 8.8811x; 1.1713x over previous
import jax
import jax.numpy as jnp
from jax import lax
from jax.experimental import pallas as pl
from jax.experimental.pallas import tpu as pltpu
from jax.experimental.pallas import tpu_sc as plsc

N = 10000
E = 320000
G = 128
D = 128
C = 16
EPS = 1e-5

NW = 32
N_PAD = 10240
CHUNK = 128
E_PAD = 327680
CPW = E_PAD // (NW * CHUNK)
EPW = E_PAD // NW
RPT = N_PAD // 16
SINK = N
RPW = N_PAD // NW
GP = G + 1

BLK = 400
NB = N // BLK

_mesh = plsc.VectorSubcoreMesh(core_axis_name="c", subcore_axis_name="s")
_sc_params = pltpu.CompilerParams(needs_layout_passes=False)


def _sc_degree_body(dst_hbm, out_hbm, dst_v, acc_v):
    c = lax.axis_index("c")
    s = lax.axis_index("s")
    wid = s * 2 + c
    pltpu.sync_copy(dst_hbm.at[wid], dst_v)
    zero16 = jnp.zeros((16,), jnp.float32)

    def zbody(i, carry):
        acc_v[pl.ds(i * 16, 16)] = zero16
        return carry

    lax.fori_loop(0, N_PAD // 16, zbody, 0)
    one16 = jnp.ones((16,), jnp.float32)

    def body(i, carry):
        idx = dst_v[pl.ds(i * 16, 16)]
        plsc.addupdate_scatter(acc_v, [idx], one16)
        return carry

    lax.fori_loop(0, EPW // 16, body, 0)
    pltpu.sync_copy(acc_v, out_hbm.at[wid])


_sc_degree = pl.kernel(
    _sc_degree_body,
    out_type=jax.ShapeDtypeStruct((NW, N_PAD), jnp.float32),
    mesh=_mesh,
    compiler_params=_sc_params,
    scratch_types=[
        pltpu.VMEM((EPW,), jnp.int32),
        pltpu.VMEM((N_PAD,), jnp.float32),
    ],
)


def _sc_propagate_body(hs_hbm, src_hbm, dst_hbm, zero_hbm, out_hbm,
                       srcA, srcB, dstA, dstB, rowsA, rowsB, acc_sh,
                       semA, semB):
    c = lax.axis_index("c")
    s = lax.axis_index("s")
    wid = s * 2 + c
    pltpu.sync_copy(zero_hbm, rowsA)

    def zbody(k, carry):
        pltpu.sync_copy(rowsA, acc_sh.at[pl.ds(s * RPT + k * CHUNK, CHUNK)])
        return carry

    lax.fori_loop(0, RPT // CHUNK, zbody, 0)
    plsc.subcore_barrier()

    pltpu.sync_copy(src_hbm.at[wid, 0], srcA)
    pltpu.sync_copy(dst_hbm.at[wid, 0], dstA)
    pltpu.async_copy(hs_hbm.at[srcA], rowsA, semA)

    def body(t, carry):
        j1 = 2 * t + 1
        pltpu.sync_copy(src_hbm.at[wid, j1], srcB)
        pltpu.sync_copy(dst_hbm.at[wid, j1], dstB)
        pltpu.make_async_copy(hs_hbm.at[srcA], rowsA, semA).wait()
        pltpu.async_copy(hs_hbm.at[srcB], rowsB, semB)
        pltpu.sync_copy(rowsA, acc_sh.at[dstA], add=True)

        @pl.when(t < CPW // 2 - 1)
        def _():
            pltpu.sync_copy(src_hbm.at[wid, j1 + 1], srcA)
            pltpu.sync_copy(dst_hbm.at[wid, j1 + 1], dstA)

        pltpu.make_async_copy(hs_hbm.at[srcB], rowsB, semB).wait()

        @pl.when(t < CPW // 2 - 1)
        def _():
            pltpu.async_copy(hs_hbm.at[srcA], rowsA, semA)

        pltpu.sync_copy(rowsB, acc_sh.at[dstB], add=True)
        return carry

    lax.fori_loop(0, CPW // 2, body, 0)
    plsc.subcore_barrier()

    def obody(k, carry):
        pltpu.sync_copy(acc_sh.at[pl.ds(s * RPT + k * CHUNK, CHUNK)], rowsA)
        pltpu.sync_copy(rowsA, out_hbm.at[c, s, pl.ds(k * CHUNK, CHUNK)])
        return carry

    lax.fori_loop(0, RPT // CHUNK, obody, 0)


_sc_propagate = pl.kernel(
    _sc_propagate_body,
    out_type=jax.ShapeDtypeStruct((2, 16, RPT, D), jnp.float32),
    mesh=_mesh,
    compiler_params=_sc_params,
    scratch_types=[
        pltpu.VMEM((CHUNK,), jnp.int32),
        pltpu.VMEM((CHUNK,), jnp.int32),
        pltpu.VMEM((CHUNK,), jnp.int32),
        pltpu.VMEM((CHUNK,), jnp.int32),
        pltpu.VMEM((CHUNK, D), jnp.float32),
        pltpu.VMEM((CHUNK, D), jnp.float32),
        pltpu.VMEM_SHARED((N_PAD, D), jnp.float32),
        pltpu.SemaphoreType.DMA,
        pltpu.SemaphoreType.DMA,
    ],
)


def _sc_segmax_body(t_hbm, b_hbm, sc_hbm, sh_hbm, out_hbm,
                    t_v, b_v, sc_v, sh_v, acc_v):
    c = lax.axis_index("c")
    s = lax.axis_index("s")
    wid = s * 2 + c
    pltpu.sync_copy(t_hbm.at[wid], t_v)
    pltpu.sync_copy(b_hbm.at[wid], b_v)
    pltpu.sync_copy(sc_hbm, sc_v)
    pltpu.sync_copy(sh_hbm, sh_v)
    ninf = jnp.full((16,), -jnp.inf, jnp.float32)

    def zbody(i, carry):
        acc_v[pl.ds(i * 16, 16)] = ninf
        return carry

    lax.fori_loop(0, GP * D // 16, zbody, 0)
    lanes = lax.iota(jnp.int32, 16)

    def group(i, carry):
        gv = b_v[pl.ds(i * 16, 16)]
        for k in range(16):
            gi = jnp.max(jnp.where(lanes == k, gv, 0))
            for j in range(8):
                off = gi * D + j * 16
                tv = t_v[pl.ds(i * (16 * D) + k * D + j * 16, 16)]
                u = jnp.maximum(
                    tv * sc_v[pl.ds(j * 16, 16)] + sh_v[pl.ds(j * 16, 16)],
                    0.0)
                acc_v[pl.ds(off, 16)] = jnp.maximum(acc_v[pl.ds(off, 16)], u)
        return carry

    lax.fori_loop(0, RPW // 16, group, 0)
    pltpu.sync_copy(acc_v, out_hbm.at[wid])


_sc_segmax = pl.kernel(
    _sc_segmax_body,
    out_type=jax.ShapeDtypeStruct((NW, GP * D), jnp.float32),
    mesh=_mesh,
    compiler_params=_sc_params,
    scratch_types=[
        pltpu.VMEM((RPW * D,), jnp.float32),
        pltpu.VMEM((RPW,), jnp.int32),
        pltpu.VMEM((D,), jnp.float32),
        pltpu.VMEM((D,), jnp.float32),
        pltpu.VMEM((GP * D,), jnp.float32),
    ],
)


def _tc_scale_mm_body(x_ref, w_ref, deg_ref, hs_ref, dinv_ref):
    h = jnp.dot(x_ref[...], w_ref[...], preferred_element_type=jnp.float32)
    deg = jnp.sum(deg_ref[...], axis=1) + 1.0
    dinv = lax.rsqrt(deg)[:, None]
    hs_ref[...] = h * dinv
    dinv_ref[...] = jnp.broadcast_to(dinv, (BLK, D))


def _tc_scale_mm(x, w, deg_parts_t):
    return pl.pallas_call(
        _tc_scale_mm_body,
        grid=(NB,),
        in_specs=[
            pl.BlockSpec((BLK, D), lambda i: (i, 0)),
            pl.BlockSpec((D, D), lambda i: (0, 0)),
            pl.BlockSpec((BLK, NW), lambda i: (i, 0)),
        ],
        out_specs=[
            pl.BlockSpec((BLK, D), lambda i: (i, 0)),
            pl.BlockSpec((BLK, D), lambda i: (i, 0)),
        ],
        out_shape=[
            jax.ShapeDtypeStruct((N, D), jnp.float32),
            jax.ShapeDtypeStruct((N, D), jnp.float32),
        ],
    )(x, w, deg_parts_t)


def _tc_combine_body(acc_ref, hs_ref, dinv_ref, b_ref, g_ref, be_ref,
                     t_ref, sum_ref, sq_ref, scale_ref, shift_ref):
    t = dinv_ref[...] * (acc_ref[0] + acc_ref[1] + hs_ref[...]) + b_ref[...]
    t_ref[...] = t

    @pl.when(pl.program_id(0) == 0)
    def _():
        sum_ref[...] = jnp.zeros_like(sum_ref)
        sq_ref[...] = jnp.zeros_like(sq_ref)

    sum_ref[...] += jnp.sum(t, axis=0, keepdims=True)
    sq_ref[...] += jnp.sum(t * t, axis=0, keepdims=True)
    mean = sum_ref[...] / N
    var = sq_ref[...] / N - mean * mean
    scale = g_ref[...] * lax.rsqrt(var + EPS)
    scale_ref[...] = scale
    shift_ref[...] = be_ref[...] - mean * scale


def _tc_combine(acc, hs, dinv, b, g, be):
    return pl.pallas_call(
        _tc_combine_body,
        grid=(NB,),
        in_specs=[
            pl.BlockSpec((2, BLK, D), lambda i: (0, i, 0)),
            pl.BlockSpec((BLK, D), lambda i: (i, 0)),
            pl.BlockSpec((BLK, D), lambda i: (i, 0)),
            pl.BlockSpec((1, D), lambda i: (0, 0)),
            pl.BlockSpec((1, D), lambda i: (0, 0)),
            pl.BlockSpec((1, D), lambda i: (0, 0)),
        ],
        out_specs=[
            pl.BlockSpec((BLK, D), lambda i: (i, 0)),
            pl.BlockSpec((1, D), lambda i: (0, 0)),
            pl.BlockSpec((1, D), lambda i: (0, 0)),
            pl.BlockSpec((1, D), lambda i: (0, 0)),
            pl.BlockSpec((1, D), lambda i: (0, 0)),
        ],
        out_shape=[
            jax.ShapeDtypeStruct((N_PAD, D), jnp.float32),
            jax.ShapeDtypeStruct((1, D), jnp.float32),
            jax.ShapeDtypeStruct((1, D), jnp.float32),
            jax.ShapeDtypeStruct((1, D), jnp.float32),
            jax.ShapeDtypeStruct((1, D), jnp.float32),
        ],
    )(acc, hs, dinv, b.reshape(1, D), g.reshape(1, D), be.reshape(1, D))


def _tc_bn_mm_body(t_ref, scale_ref, shift_ref, w_ref, dinv_ref, out_ref):
    u = jnp.maximum(t_ref[...] * scale_ref[...] + shift_ref[...], 0.0)
    out_ref[...] = jnp.dot(u, w_ref[...],
                           preferred_element_type=jnp.float32) * dinv_ref[...]


def _tc_bn_mm(t, scale, shift, w, dinv):
    return pl.pallas_call(
        _tc_bn_mm_body,
        grid=(NB,),
        in_specs=[
            pl.BlockSpec((BLK, D), lambda i: (i, 0)),
            pl.BlockSpec((1, D), lambda i: (0, 0)),
            pl.BlockSpec((1, D), lambda i: (0, 0)),
            pl.BlockSpec((D, D), lambda i: (0, 0)),
            pl.BlockSpec((BLK, D), lambda i: (i, 0)),
        ],
        out_specs=pl.BlockSpec((BLK, D), lambda i: (i, 0)),
        out_shape=jax.ShapeDtypeStruct((N, D), jnp.float32),
    )(t, scale, shift, w, dinv)


def _tc_head_body(p_ref, wl1_ref, bl1_ref, g_ref, be_ref, wl2_ref, bl2_ref,
                  out_ref):
    p = p_ref[...]
    pm = p[0]
    for w in range(1, NW):
        pm = jnp.maximum(pm, p[w])
    pooled = pm[:G, :]
    z = jnp.dot(pooled, wl1_ref[...],
                preferred_element_type=jnp.float32) + bl1_ref[...]
    mean = jnp.mean(z, axis=0, keepdims=True)
    var = jnp.mean((z - mean) * (z - mean), axis=0, keepdims=True)
    zb = jnp.maximum((z - mean) * lax.rsqrt(var + EPS) * g_ref[...]
                     + be_ref[...], 0.0)
    z2 = jnp.dot(zb, wl2_ref[...],
                 preferred_element_type=jnp.float32) + bl2_ref[...]
    m = jnp.max(z2, axis=1, keepdims=True)
    lse = jnp.log(jnp.sum(jnp.exp(z2 - m), axis=1, keepdims=True)) + m
    out_ref[...] = z2 - lse


def _tc_head(parts, wl1, bl1, g3, be3, wl2, bl2):
    return pl.pallas_call(
        _tc_head_body,
        out_shape=jax.ShapeDtypeStruct((G, C), jnp.float32),
    )(parts, wl1, bl1.reshape(1, D), g3.reshape(1, D), be3.reshape(1, D),
      wl2, bl2.reshape(1, C))


def kernel(x, edge_index, batch, W1, b1, g1, be1, W2, b2, g2, be2,
           Wl1, bl1, g3, be3, Wl2, bl2):
    src = edge_index[0].astype(jnp.int32)
    dst = edge_index[1].astype(jnp.int32)
    pad = E_PAD - E
    src_p = jnp.concatenate([src, jnp.zeros((pad,), jnp.int32)])
    dst_p = jnp.concatenate([dst, jnp.full((pad,), SINK, jnp.int32)])
    src3 = src_p.reshape(NW, CPW, CHUNK)
    dst3 = dst_p.reshape(NW, CPW, CHUNK)
    dst2 = dst_p.reshape(NW, EPW)
    zeros = jnp.zeros((CHUNK, D), jnp.float32)
    batch2 = jnp.concatenate([batch.astype(jnp.int32),
                              jnp.full((N_PAD - N,), G, jnp.int32)])
    batch2 = batch2.reshape(NW, RPW)

    deg_parts = _sc_degree(dst2)
    hs, dinv = _tc_scale_mm(x, W1, deg_parts.T)

    acc1 = _sc_propagate(hs, src3, dst3, zeros).reshape(2, N_PAD, D)
    t1, _, _, scale1, shift1 = _tc_combine(acc1, hs, dinv, b1, g1, be1)
    h2s = _tc_bn_mm(t1, scale1, shift1, W2, dinv)

    acc2 = _sc_propagate(h2s, src3, dst3, zeros).reshape(2, N_PAD, D)
    t2, _, _, scale2, shift2 = _tc_combine(acc2, h2s, dinv, b2, g2, be2)
    parts = _sc_segmax(t2.reshape(NW, RPW * D), batch2,
                       scale2.reshape(D), shift2.reshape(D))

    return _tc_head(parts.reshape(NW, GP, D), Wl1, bl1, g3, be3, Wl2, bl2)
